# Initial kernel scaffold; baseline (speedup 1.0000x reference)
import jax
import jax.numpy as jnp
from jax.experimental import pallas as pl


def kernel(x, indices):
    raise NotImplementedError("write your pallas kernel here")


# speedup vs baseline: 3.0722x; 3.0722x over previous
import functools

import jax
import jax.numpy as jnp
from jax import lax
from jax.experimental import pallas as pl
from jax.experimental.pallas import tpu as pltpu
from jax.experimental.pallas import tpu_sc as plsc

C = 32
NW = 32
G = 128
CH = 8
CHUNK = CH * G


def _transpose_to_rows(x2d, n):
    bk = 2048

    def body(x_ref, o_ref):
        o_ref[...] = x_ref[...].T

    return pl.pallas_call(
        body,
        grid=(n // bk,),
        in_specs=[pl.BlockSpec((C, bk), lambda i: (0, i))],
        out_specs=pl.BlockSpec((bk, C), lambda i: (i, 0)),
        out_shape=jax.ShapeDtypeStruct((n, C), jnp.float32),
    )(x2d)


def _transpose_to_cols(xt, n):
    bk = 2048

    def body(x_ref, o_ref):
        o_ref[...] = x_ref[...].T

    return pl.pallas_call(
        body,
        grid=(n // bk,),
        in_specs=[pl.BlockSpec((bk, C), lambda i: (i, 0))],
        out_specs=pl.BlockSpec((C, bk), lambda i: (0, i)),
        out_shape=jax.ShapeDtypeStruct((C, n), jnp.float32),
    )(xt)


def _sc_scatter_rows(xt, idx2d, n):
    rows_per_w = n // NW
    iters = rows_per_w // CHUNK
    mesh = plsc.VectorSubcoreMesh(core_axis_name="c", subcore_axis_name="s")

    @functools.partial(
        pl.kernel,
        mesh=mesh,
        compiler_params=pltpu.CompilerParams(use_tc_tiling_on_sc=False),
        out_type=jax.ShapeDtypeStruct((n, C), jnp.float32),
        scratch_types=[
            pltpu.VMEM((CH, G), jnp.int32),
            pltpu.VMEM((CHUNK, C), jnp.float32),
            pltpu.SemaphoreType.DMA,
        ],
    )
    def scatter_kernel(xt_hbm, idx_hbm, out_hbm, idx_v, rows_v, sem):
        wid = lax.axis_index("s") * 2 + lax.axis_index("c")
        base = wid * rows_per_w

        def step(it, carry):
            rbase = pl.multiple_of(base + it * CHUNK, CHUNK)
            pltpu.sync_copy(
                idx_hbm.at[pl.ds(pl.multiple_of(rbase // G, CH), CH)], idx_v
            )
            pltpu.sync_copy(xt_hbm.at[pl.ds(rbase, CHUNK)], rows_v)
            descs = [
                pltpu.async_copy(
                    rows_v.at[pl.ds(g * G, G)], out_hbm.at[idx_v.at[g]], sem
                )
                for g in range(CH)
            ]
            for d in descs:
                d.wait()
            return carry

        lax.fori_loop(0, iters, step, 0)

    return scatter_kernel(xt, idx2d)


def kernel(x, indices):
    b, c, n = x.shape
    x2d = x.reshape(c, n)
    idx2d = indices.reshape(n // G, G)
    xt = _transpose_to_rows(x2d, n)
    out_t = _sc_scatter_rows(xt, idx2d, n)
    out2d = _transpose_to_cols(out_t, n)
    return out2d.reshape(b, c, n)


# speedup vs baseline: 3.1426x; 1.0229x over previous
import functools

import jax
import jax.numpy as jnp
from jax import lax
from jax.experimental import pallas as pl
from jax.experimental.pallas import tpu as pltpu
from jax.experimental.pallas import tpu_sc as plsc

C = 32
NW = 32
G = 128
CH = 8
CHUNK = CH * G
NBUF = 2


def _transpose_to_rows(x2d, n):
    bk = 2048

    def body(x_ref, o_ref):
        o_ref[...] = x_ref[...].T

    return pl.pallas_call(
        body,
        grid=(n // bk,),
        in_specs=[pl.BlockSpec((C, bk), lambda i: (0, i))],
        out_specs=pl.BlockSpec((bk, C), lambda i: (i, 0)),
        out_shape=jax.ShapeDtypeStruct((n, C), jnp.float32),
    )(x2d)


def _transpose_to_cols(xt, n):
    bk = 2048

    def body(x_ref, o_ref):
        o_ref[...] = x_ref[...].T

    return pl.pallas_call(
        body,
        grid=(n // bk,),
        in_specs=[pl.BlockSpec((bk, C), lambda i: (i, 0))],
        out_specs=pl.BlockSpec((C, bk), lambda i: (0, i)),
        out_shape=jax.ShapeDtypeStruct((C, n), jnp.float32),
    )(xt)


def _sc_scatter_rows(xt, idx2d, n):
    rows_per_w = n // NW
    iters = rows_per_w // CHUNK
    groups_per_w = rows_per_w // G
    mesh = plsc.VectorSubcoreMesh(core_axis_name="c", subcore_axis_name="s")

    @functools.partial(
        pl.kernel,
        mesh=mesh,
        compiler_params=pltpu.CompilerParams(use_tc_tiling_on_sc=False),
        out_type=jax.ShapeDtypeStruct((n, C), jnp.float32),
        scratch_types=[
            pltpu.VMEM((groups_per_w, G), jnp.int32),
            [pltpu.VMEM((CHUNK, C), jnp.float32) for _ in range(NBUF)],
            pltpu.SemaphoreType.DMA,
            [pltpu.SemaphoreType.DMA for _ in range(NBUF)],
            [pltpu.SemaphoreType.DMA for _ in range(NBUF)],
        ],
    )
    def scatter_kernel(xt_hbm, idx_hbm, out_hbm, idx_v, row_bufs, isem, lsem, ssem):
        wid = lax.axis_index("s") * 2 + lax.axis_index("c")
        base = wid * rows_per_w

        pltpu.async_copy(
            idx_hbm.at[pl.ds(pl.multiple_of(base // G, groups_per_w), groups_per_w)],
            idx_v,
            isem,
        ).wait()

        def issue_load(it, b):
            rbase = pl.multiple_of(base + it * CHUNK, CHUNK)
            pltpu.async_copy(xt_hbm.at[pl.ds(rbase, CHUNK)], row_bufs[b], lsem[b])

        for b in range(NBUF):
            issue_load(b, b)

        def outer(i2, carry):
            for b in range(NBUF):
                it = i2 * NBUF + b
                pltpu.make_async_copy(
                    xt_hbm.at[pl.ds(0, CHUNK)], row_bufs[b], lsem[b]
                ).wait()
                descs = [
                    pltpu.async_copy(
                        row_bufs[b].at[pl.ds(g * G, G)],
                        out_hbm.at[idx_v.at[it * CH + g]],
                        ssem[b],
                    )
                    for g in range(CH)
                ]
                for d in descs:
                    d.wait()

                @pl.when(it + NBUF < iters)
                def _():
                    issue_load(it + NBUF, b)

            return carry

        lax.fori_loop(0, iters // NBUF, outer, 0)

    return scatter_kernel(xt, idx2d)


def kernel(x, indices):
    b, c, n = x.shape
    x2d = x.reshape(c, n)
    idx2d = indices.reshape(n // G, G)
    xt = _transpose_to_rows(x2d, n)
    out_t = _sc_scatter_rows(xt, idx2d, n)
    out2d = _transpose_to_cols(out_t, n)
    return out2d.reshape(b, c, n)


# speedup vs baseline: 5.3019x; 1.6871x over previous
import functools

import jax
import jax.numpy as jnp
from jax import lax
from jax.experimental import pallas as pl
from jax.experimental.pallas import tpu as pltpu
from jax.experimental.pallas import tpu_sc as plsc

C = 32
NW = 32
G = 128
CH = 8
CHUNK = CH * G
NBUF = 2
BKQ = 1024


def _transpose_to_rows(x2d, n):
    n4 = n // 4
    nb = n4 // BKQ

    def body(x0, x1, x2, x3, o_ref):
        for k, xk in enumerate((x0, x1, x2, x3)):
            o_ref[:, 32 * k:32 * (k + 1)] = xk[...].T

    return pl.pallas_call(
        body,
        grid=(nb,),
        in_specs=[
            pl.BlockSpec((C, BKQ), lambda i, k=k: (0, k * nb + i))
            for k in range(4)
        ],
        out_specs=pl.BlockSpec((BKQ, 4 * C), lambda i: (i, 0)),
        out_shape=jax.ShapeDtypeStruct((n4, 4 * C), jnp.float32),
    )(x2d, x2d, x2d, x2d)


def _transpose_to_cols(xt4, n):
    n4 = n // 4
    nb = n4 // BKQ

    def body(x_ref, o_ref):
        q = pl.program_id(1)
        for k in range(4):
            @pl.when(q == k)
            def _():
                o_ref[...] = x_ref[:, 32 * k:32 * (k + 1)].T

    return pl.pallas_call(
        body,
        grid=(nb, 4),
        in_specs=[pl.BlockSpec((BKQ, 4 * C), lambda u, q: (u, 0))],
        out_specs=pl.BlockSpec((C, BKQ), lambda u, q: (0, q * nb + u)),
        out_shape=jax.ShapeDtypeStruct((C, n), jnp.float32),
    )(xt4)


def _sc_scatter_rows(xt, idx16, n):
    n4 = n // 4
    rows_per_w = n // NW
    iters = rows_per_w // CHUNK
    u_per_w = rows_per_w // 4
    slab_rows = u_per_w // 16
    mesh = plsc.VectorSubcoreMesh(core_axis_name="c", subcore_axis_name="s")

    @functools.partial(
        pl.kernel,
        mesh=mesh,
        compiler_params=pltpu.CompilerParams(
            use_tc_tiling_on_sc=False, needs_layout_passes=False),
        out_type=jax.ShapeDtypeStruct((n, C), jnp.float32),
        scratch_types=[
            pltpu.VMEM((4 * slab_rows, 16), jnp.int32),
            [[pltpu.VMEM((G,), jnp.int32) for _ in range(CH)]
             for _ in range(NBUF)],
            [pltpu.VMEM((CHUNK, C), jnp.float32) for _ in range(NBUF)],
            pltpu.SemaphoreType.DMA,
            [pltpu.SemaphoreType.DMA for _ in range(NBUF)],
            [pltpu.SemaphoreType.DMA for _ in range(NBUF)],
        ],
    )
    def scatter_kernel(xt_hbm, idx_hbm, out_hbm, idx_v, i_bufs, row_bufs,
                       isem, lsem, ssem):
        wid = lax.axis_index("s") * 2 + lax.axis_index("c")
        base = wid * rows_per_w

        for q in range(4):
            src_row = pl.multiple_of(
                q * (n4 // 16) + wid * slab_rows, slab_rows)
            pltpu.async_copy(
                idx_hbm.at[pl.ds(src_row, slab_rows)],
                idx_v.at[pl.ds(q * slab_rows, slab_rows)],
                isem,
            )
        pltpu.make_async_copy(
            idx_hbm.at[pl.ds(0, 4 * slab_rows)], idx_v, isem
        ).wait()

        lane = lax.iota(jnp.int32, 16)
        sh = n4.bit_length() - 1

        def issue_load(it, b):
            rbase = pl.multiple_of(base + it * CHUNK, CHUNK)
            pltpu.async_copy(xt_hbm.at[pl.ds(rbase, CHUNK)], row_bufs[b], lsem[b])

        for b in range(NBUF):
            issue_load(b, b)

        def outer(i2, carry):
            for b in range(NBUF):
                it = i2 * NBUF + b
                for g in range(CH):
                    gi = it * CH + g
                    for q in range(4):
                        for h in range(2):
                            p = idx_v[slab_rows * q + 2 * gi + h, :]
                            wv = (p & (n4 - 1)) * 4 + lax.shift_right_logical(p, sh)
                            plsc.store_scatter(
                                i_bufs[b][g], [lane * 4 + (64 * h + q)], wv
                            )
                pltpu.make_async_copy(
                    xt_hbm.at[pl.ds(0, CHUNK)], row_bufs[b], lsem[b]
                ).wait()
                descs = [
                    pltpu.async_copy(
                        row_bufs[b].at[pl.ds(g * G, G)],
                        out_hbm.at[i_bufs[b][g]],
                        ssem[b],
                    )
                    for g in range(CH)
                ]
                for d in descs:
                    d.wait()

                @pl.when(it + NBUF < iters)
                def _():
                    issue_load(it + NBUF, b)

            return carry

        lax.fori_loop(0, iters // NBUF, outer, 0)

    return scatter_kernel(xt, idx16)


def kernel(x, indices):
    b, c, n = x.shape
    x2d = x.reshape(c, n)
    idx16 = indices.reshape(n // 16, 16)
    xt4 = _transpose_to_rows(x2d, n)
    out_t = _sc_scatter_rows(xt4.reshape(n, C), idx16, n)
    out2d = _transpose_to_cols(out_t.reshape(n // 4, 4 * C), n)
    return out2d.reshape(b, c, n)


# speedup vs baseline: 8.7859x; 1.6571x over previous
import functools

import jax
import jax.numpy as jnp
from jax import lax
from jax.experimental import pallas as pl
from jax.experimental.pallas import tpu as pltpu
from jax.experimental.pallas import tpu_sc as plsc

C = 32
NW = 32
G = 128
CH = 8
CHUNK = CH * G
NBUF = 2
BKQ = 2048


def _eye128():
    r = lax.broadcasted_iota(jnp.int32, (4 * C, 4 * C), 0)
    c = lax.broadcasted_iota(jnp.int32, (4 * C, 4 * C), 1)
    return (r == c).astype(jnp.float32)


def _transpose_to_rows(x2d, n):
    n4 = n // 4
    nb = n4 // BKQ

    def body(x0, x1, x2, x3, o_ref):
        x = jnp.concatenate([x0[...], x1[...], x2[...], x3[...]], axis=0)
        o_ref[...] = lax.dot_general(
            x, _eye128(), (((0,), (0,)), ((), ())),
            preferred_element_type=jnp.float32)

    return pl.pallas_call(
        body,
        grid=(nb,),
        in_specs=[
            pl.BlockSpec((C, BKQ), lambda i, k=k: (0, k * nb + i))
            for k in range(4)
        ],
        out_specs=pl.BlockSpec((BKQ, 4 * C), lambda i: (i, 0)),
        out_shape=jax.ShapeDtypeStruct((n4, 4 * C), jnp.float32),
    )(x2d, x2d, x2d, x2d)


def _transpose_to_cols(xt4, n):
    n4 = n // 4
    nb = n4 // BKQ

    def body(x_ref, o_ref):
        q = pl.program_id(1)
        r = lax.broadcasted_iota(jnp.int32, (C, 4 * C), 0)
        l = lax.broadcasted_iota(jnp.int32, (C, 4 * C), 1)
        sel = (l == C * q + r).astype(jnp.float32)
        o_ref[...] = lax.dot_general(
            sel, x_ref[...], (((1,), (1,)), ((), ())),
            preferred_element_type=jnp.float32)

    return pl.pallas_call(
        body,
        grid=(nb, 4),
        in_specs=[pl.BlockSpec((BKQ, 4 * C), lambda u, q: (u, 0))],
        out_specs=pl.BlockSpec((C, BKQ), lambda u, q: (0, q * nb + u)),
        out_shape=jax.ShapeDtypeStruct((C, n), jnp.float32),
    )(xt4)


def _sc_scatter_rows(xt, idx16, n):
    n4 = n // 4
    rows_per_w = n // NW
    iters = rows_per_w // CHUNK
    u_per_w = rows_per_w // 4
    slab_rows = u_per_w // 16
    mesh = plsc.VectorSubcoreMesh(core_axis_name="c", subcore_axis_name="s")

    @functools.partial(
        pl.kernel,
        mesh=mesh,
        compiler_params=pltpu.CompilerParams(
            use_tc_tiling_on_sc=False, needs_layout_passes=False),
        out_type=jax.ShapeDtypeStruct((n, C), jnp.float32),
        scratch_types=[
            pltpu.VMEM((4 * slab_rows, 16), jnp.int32),
            [[pltpu.VMEM((G,), jnp.int32) for _ in range(CH)]
             for _ in range(NBUF)],
            [pltpu.VMEM((CHUNK, C), jnp.float32) for _ in range(NBUF)],
            pltpu.SemaphoreType.DMA,
            [pltpu.SemaphoreType.DMA for _ in range(NBUF)],
            [pltpu.SemaphoreType.DMA for _ in range(NBUF)],
        ],
    )
    def scatter_kernel(xt_hbm, idx_hbm, out_hbm, idx_v, i_bufs, row_bufs,
                       isem, lsem, ssem):
        wid = lax.axis_index("s") * 2 + lax.axis_index("c")
        base = wid * rows_per_w

        for q in range(4):
            src_row = pl.multiple_of(
                q * (n4 // 16) + wid * slab_rows, slab_rows)
            pltpu.async_copy(
                idx_hbm.at[pl.ds(src_row, slab_rows)],
                idx_v.at[pl.ds(q * slab_rows, slab_rows)],
                isem,
            )
        pltpu.make_async_copy(
            idx_hbm.at[pl.ds(0, 4 * slab_rows)], idx_v, isem
        ).wait()

        lane = lax.iota(jnp.int32, 16)
        sh = n4.bit_length() - 1

        def issue_load(it, b):
            rbase = pl.multiple_of(base + it * CHUNK, CHUNK)
            pltpu.async_copy(xt_hbm.at[pl.ds(rbase, CHUNK)], row_bufs[b], lsem[b])

        for b in range(NBUF):
            issue_load(b, b)

        def outer(i2, carry):
            for b in range(NBUF):
                it = i2 * NBUF + b
                for g in range(CH):
                    gi = it * CH + g
                    for q in range(4):
                        for h in range(2):
                            p = idx_v[slab_rows * q + 2 * gi + h, :]
                            wv = (p & (n4 - 1)) * 4 + lax.shift_right_logical(p, sh)
                            plsc.store_scatter(
                                i_bufs[b][g], [lane * 4 + (64 * h + q)], wv
                            )
                pltpu.make_async_copy(
                    xt_hbm.at[pl.ds(0, CHUNK)], row_bufs[b], lsem[b]
                ).wait()
                descs = [
                    pltpu.async_copy(
                        row_bufs[b].at[pl.ds(g * G, G)],
                        out_hbm.at[i_bufs[b][g]],
                        ssem[b],
                    )
                    for g in range(CH)
                ]
                for d in descs:
                    d.wait()

                @pl.when(it + NBUF < iters)
                def _():
                    issue_load(it + NBUF, b)

            return carry

        lax.fori_loop(0, iters // NBUF, outer, 0)

    return scatter_kernel(xt, idx16)


def kernel(x, indices):
    b, c, n = x.shape
    x2d = x.reshape(c, n)
    idx16 = indices.reshape(n // 16, 16)
    xt4 = _transpose_to_rows(x2d, n)
    out_t = _sc_scatter_rows(xt4.reshape(n, C), idx16, n)
    out2d = _transpose_to_cols(out_t.reshape(n // 4, 4 * C), n)
    return out2d.reshape(b, c, n)


# speedup vs baseline: 11.0372x; 1.2562x over previous
import functools

import jax
import jax.numpy as jnp
from jax import lax
from jax.experimental import pallas as pl
from jax.experimental.pallas import tpu as pltpu
from jax.experimental.pallas import tpu_sc as plsc

C = 32
NW = 32
G = 128
CH = 8
CHUNK = CH * G
NBUF = 2
BKQ = 2048


def _eye128():
    r = lax.broadcasted_iota(jnp.int32, (4 * C, 4 * C), 0)
    c = lax.broadcasted_iota(jnp.int32, (4 * C, 4 * C), 1)
    return (r == c).astype(jnp.float32)


def _transpose_to_rows(x2d, n):
    n4 = n // 4
    nb = n4 // BKQ

    def body(x0, x1, x2, x3, o_ref):
        x = jnp.concatenate([x0[...], x1[...], x2[...], x3[...]], axis=0)
        o_ref[...] = lax.dot_general(
            x, _eye128(), (((0,), (0,)), ((), ())),
            preferred_element_type=jnp.float32)

    return pl.pallas_call(
        body,
        grid=(nb,),
        in_specs=[
            pl.BlockSpec((C, BKQ), lambda i, k=k: (0, k * nb + i))
            for k in range(4)
        ],
        out_specs=pl.BlockSpec((BKQ, 4 * C), lambda i: (i, 0)),
        out_shape=jax.ShapeDtypeStruct((n4, 4 * C), jnp.float32),
    )(x2d, x2d, x2d, x2d)


P = 1024


def _transpose_to_cols(ot4, n):
    n4 = n // 4
    nb = n4 // P

    def body(x_ref, o_ref):
        r = lax.broadcasted_iota(jnp.int32, (C, 4 * C), 0)
        l = lax.broadcasted_iota(jnp.int32, (C, 4 * C), 1)
        for k in range(4):
            sel = (l == C * k + r).astype(jnp.float32)
            o_ref[:, P * k:P * (k + 1)] = lax.dot_general(
                sel, x_ref[...], (((1,), (1,)), ((), ())),
                preferred_element_type=jnp.float32)

    return pl.pallas_call(
        body,
        grid=(nb,),
        in_specs=[pl.BlockSpec((P, 4 * C), lambda i: (i, 0))],
        out_specs=pl.BlockSpec((C, 4 * P), lambda i: (0, i)),
        out_shape=jax.ShapeDtypeStruct((C, n), jnp.float32),
    )(ot4)


def _sc_scatter_rows(xt, idx16, n):
    n4 = n // 4
    rows_per_w = n // NW
    iters = rows_per_w // CHUNK
    u_per_w = rows_per_w // 4
    slab_rows = u_per_w // 16
    mesh = plsc.VectorSubcoreMesh(core_axis_name="c", subcore_axis_name="s")

    @functools.partial(
        pl.kernel,
        mesh=mesh,
        compiler_params=pltpu.CompilerParams(
            use_tc_tiling_on_sc=False, needs_layout_passes=False),
        out_type=jax.ShapeDtypeStruct((n, C), jnp.float32),
        scratch_types=[
            pltpu.VMEM((4 * slab_rows, 16), jnp.int32),
            [[pltpu.VMEM((G,), jnp.int32) for _ in range(CH)]
             for _ in range(NBUF)],
            [pltpu.VMEM((CHUNK, C), jnp.float32) for _ in range(NBUF)],
            pltpu.SemaphoreType.DMA,
            [pltpu.SemaphoreType.DMA for _ in range(NBUF)],
            [pltpu.SemaphoreType.DMA for _ in range(NBUF)],
        ],
    )
    def scatter_kernel(xt_hbm, idx_hbm, out_hbm, idx_v, i_bufs, row_bufs,
                       isem, lsem, ssem):
        wid = lax.axis_index("s") * 2 + lax.axis_index("c")
        base = wid * rows_per_w

        for q in range(4):
            src_row = pl.multiple_of(
                q * (n4 // 16) + wid * slab_rows, slab_rows)
            pltpu.async_copy(
                idx_hbm.at[pl.ds(src_row, slab_rows)],
                idx_v.at[pl.ds(q * slab_rows, slab_rows)],
                isem,
            )
        pltpu.make_async_copy(
            idx_hbm.at[pl.ds(0, 4 * slab_rows)], idx_v, isem
        ).wait()

        lane = lax.iota(jnp.int32, 16)

        def issue_load(it, b):
            rbase = pl.multiple_of(base + it * CHUNK, CHUNK)
            pltpu.async_copy(xt_hbm.at[pl.ds(rbase, CHUNK)], row_bufs[b], lsem[b])

        for b in range(NBUF):
            issue_load(b, b)

        def outer(i2, carry):
            for b in range(NBUF):
                it = i2 * NBUF + b
                for g in range(CH):
                    gi = it * CH + g
                    for q in range(4):
                        for h in range(2):
                            p = idx_v[slab_rows * q + 2 * gi + h, :]
                            wv = (
                                (lax.shift_right_logical(p, 12) * 4096)
                                | ((p & (P - 1)) * 4)
                                | (lax.shift_right_logical(p, 10) & 3)
                            )
                            plsc.store_scatter(
                                i_bufs[b][g], [lane * 4 + (64 * h + q)], wv
                            )
                pltpu.make_async_copy(
                    xt_hbm.at[pl.ds(0, CHUNK)], row_bufs[b], lsem[b]
                ).wait()
                descs = [
                    pltpu.async_copy(
                        row_bufs[b].at[pl.ds(g * G, G)],
                        out_hbm.at[i_bufs[b][g]],
                        ssem[b],
                    )
                    for g in range(CH)
                ]
                for d in descs:
                    d.wait()

                @pl.when(it + NBUF < iters)
                def _():
                    issue_load(it + NBUF, b)

            return carry

        lax.fori_loop(0, iters // NBUF, outer, 0)

    return scatter_kernel(xt, idx16)


def kernel(x, indices):
    b, c, n = x.shape
    x2d = x.reshape(c, n)
    idx16 = indices.reshape(n // 16, 16)
    xt4 = _transpose_to_rows(x2d, n)
    out_t = _sc_scatter_rows(xt4.reshape(n, C), idx16, n)
    out2d = _transpose_to_cols(out_t.reshape(n // 4, 4 * C), n)
    return out2d.reshape(b, c, n)


# speedup vs baseline: 12.7569x; 1.1558x over previous
import functools

import jax
import jax.numpy as jnp
from jax import lax
from jax.experimental import pallas as pl
from jax.experimental.pallas import tpu as pltpu
from jax.experimental.pallas import tpu_sc as plsc

C = 32
NW = 32
G = 128
CH = 8
CHUNK = CH * G
NBUF = 2
BKQ = 2048


def _eye128():
    r = lax.broadcasted_iota(jnp.int32, (4 * C, 4 * C), 0)
    c = lax.broadcasted_iota(jnp.int32, (4 * C, 4 * C), 1)
    return (r == c).astype(jnp.float32)


def _transpose_to_rows(x2d, n):
    n4 = n // 4
    nb = n4 // BKQ

    def body(x0, x1, x2, x3, o_ref):
        x = jnp.concatenate([x0[...], x1[...], x2[...], x3[...]], axis=0)
        o_ref[...] = lax.dot_general(
            x, _eye128(), (((0,), (0,)), ((), ())),
            preferred_element_type=jnp.float32)

    return pl.pallas_call(
        body,
        grid=(nb,),
        in_specs=[
            pl.BlockSpec((C, BKQ), lambda i, k=k: (0, k * nb + i))
            for k in range(4)
        ],
        out_specs=pl.BlockSpec((BKQ, 4 * C), lambda i: (i, 0)),
        out_shape=jax.ShapeDtypeStruct((n4, 4 * C), jnp.float32),
    )(x2d, x2d, x2d, x2d)


P = 2048


def _transpose_to_cols(ot4, n):
    n4 = n // 4
    nb = n4 // P

    def body(x_ref, o_ref):
        r = lax.broadcasted_iota(jnp.int32, (C, 4 * C), 0)
        l = lax.broadcasted_iota(jnp.int32, (C, 4 * C), 1)
        for k in range(4):
            sel = (l == C * k + r).astype(jnp.float32)
            o_ref[:, P * k:P * (k + 1)] = lax.dot_general(
                sel, x_ref[...], (((1,), (1,)), ((), ())),
                preferred_element_type=jnp.float32)

    return pl.pallas_call(
        body,
        grid=(nb,),
        in_specs=[pl.BlockSpec((P, 4 * C), lambda i: (i, 0))],
        out_specs=pl.BlockSpec((C, 4 * P), lambda i: (0, i)),
        out_shape=jax.ShapeDtypeStruct((C, n), jnp.float32),
    )(ot4)


def _sc_scatter_rows(xt, idx16, n):
    n4 = n // 4
    rows_per_w = n // NW
    iters = rows_per_w // CHUNK
    u_per_w = rows_per_w // 4
    slab_rows = u_per_w // 16
    mesh = plsc.VectorSubcoreMesh(core_axis_name="c", subcore_axis_name="s")

    @functools.partial(
        pl.kernel,
        mesh=mesh,
        compiler_params=pltpu.CompilerParams(
            use_tc_tiling_on_sc=False, needs_layout_passes=False),
        out_type=jax.ShapeDtypeStruct((n, C), jnp.float32),
        scratch_types=[
            pltpu.VMEM((4 * slab_rows, 16), jnp.int32),
            [[pltpu.VMEM((G,), jnp.int32) for _ in range(CH)]
             for _ in range(NBUF)],
            [pltpu.VMEM((CHUNK, C), jnp.float32) for _ in range(NBUF)],
            pltpu.SemaphoreType.DMA,
            [pltpu.SemaphoreType.DMA for _ in range(NBUF)],
            [pltpu.SemaphoreType.DMA for _ in range(NBUF)],
        ],
    )
    def scatter_kernel(xt_hbm, idx_hbm, out_hbm, idx_v, i_bufs, row_bufs,
                       isem, lsem, ssem):
        wid = lax.axis_index("s") * 2 + lax.axis_index("c")
        base = wid * rows_per_w

        for q in range(4):
            src_row = pl.multiple_of(
                q * (n4 // 16) + wid * slab_rows, slab_rows)
            pltpu.async_copy(
                idx_hbm.at[pl.ds(src_row, slab_rows)],
                idx_v.at[pl.ds(q * slab_rows, slab_rows)],
                isem,
            )
        pltpu.make_async_copy(
            idx_hbm.at[pl.ds(0, 4 * slab_rows)], idx_v, isem
        ).wait()

        lane = lax.iota(jnp.int32, 16)

        def issue_load(it, b):
            rbase = pl.multiple_of(base + it * CHUNK, CHUNK)
            pltpu.async_copy(xt_hbm.at[pl.ds(rbase, CHUNK)], row_bufs[b], lsem[b])

        for b in range(NBUF):
            issue_load(b, b)

        def outer(i2, carry):
            for b in range(NBUF):
                it = i2 * NBUF + b
                for g in range(CH):
                    gi = it * CH + g
                    for q in range(4):
                        for h in range(2):
                            p = idx_v[slab_rows * q + 2 * gi + h, :]
                            lp = P.bit_length() - 1
                            wv = (
                                (p & ~(4 * P - 1))
                                | ((p & (P - 1)) * 4)
                                | (lax.shift_right_logical(p, lp) & 3)
                            )
                            plsc.store_scatter(
                                i_bufs[b][g], [lane * 4 + (64 * h + q)], wv
                            )
                pltpu.make_async_copy(
                    xt_hbm.at[pl.ds(0, CHUNK)], row_bufs[b], lsem[b]
                ).wait()
                descs = [
                    pltpu.async_copy(
                        row_bufs[b].at[pl.ds(g * G, G)],
                        out_hbm.at[i_bufs[b][g]],
                        ssem[b],
                    )
                    for g in range(CH)
                ]
                for d in descs:
                    d.wait()

                @pl.when(it + NBUF < iters)
                def _():
                    issue_load(it + NBUF, b)

            return carry

        lax.fori_loop(0, iters // NBUF, outer, 0)

    return scatter_kernel(xt, idx16)


def kernel(x, indices):
    b, c, n = x.shape
    x2d = x.reshape(c, n)
    idx16 = indices.reshape(n // 16, 16)
    xt4 = _transpose_to_rows(x2d, n)
    out_t = _sc_scatter_rows(xt4.reshape(n, C), idx16, n)
    out2d = _transpose_to_cols(out_t.reshape(n // 4, 4 * C), n)
    return out2d.reshape(b, c, n)


# speedup vs baseline: 13.9365x; 1.0925x over previous
import functools

import jax
import jax.numpy as jnp
from jax import lax
from jax.experimental import pallas as pl
from jax.experimental.pallas import tpu as pltpu
from jax.experimental.pallas import tpu_sc as plsc

C = 32
NW = 32
G = 128
CH = 8
CHUNK = CH * G
NBUF = 2
BKQ = 2048


def _eye128():
    r = lax.broadcasted_iota(jnp.int32, (4 * C, 4 * C), 0)
    c = lax.broadcasted_iota(jnp.int32, (4 * C, 4 * C), 1)
    return (r == c).astype(jnp.float32)


def _transpose_to_rows(x2d, n):
    n4 = n // 4
    nb = n4 // BKQ

    def body(x0, x1, x2, x3, o_ref):
        x = jnp.concatenate([x0[...], x1[...], x2[...], x3[...]], axis=0)
        o_ref[...] = lax.dot_general(
            x, _eye128(), (((0,), (0,)), ((), ())),
            preferred_element_type=jnp.float32)

    return pl.pallas_call(
        body,
        grid=(nb,),
        in_specs=[
            pl.BlockSpec((C, BKQ), lambda i, k=k: (0, k * nb + i))
            for k in range(4)
        ],
        out_specs=pl.BlockSpec((BKQ, 4 * C), lambda i: (i, 0)),
        out_shape=jax.ShapeDtypeStruct((n4, 4 * C), jnp.float32),
    )(x2d, x2d, x2d, x2d)


P = 4096


def _transpose_to_cols(ot4, n):
    n4 = n // 4
    nb = n4 // P

    def body(x_ref, o_ref):
        r = lax.broadcasted_iota(jnp.int32, (C, 4 * C), 0)
        l = lax.broadcasted_iota(jnp.int32, (C, 4 * C), 1)
        for k in range(4):
            sel = (l == C * k + r).astype(jnp.float32)
            o_ref[:, P * k:P * (k + 1)] = lax.dot_general(
                sel, x_ref[...], (((1,), (1,)), ((), ())),
                preferred_element_type=jnp.float32)

    return pl.pallas_call(
        body,
        grid=(nb,),
        in_specs=[pl.BlockSpec((P, 4 * C), lambda i: (i, 0))],
        out_specs=pl.BlockSpec((C, 4 * P), lambda i: (0, i)),
        out_shape=jax.ShapeDtypeStruct((C, n), jnp.float32),
    )(ot4)


def _sc_scatter_rows(xt, idx16, n):
    n4 = n // 4
    rows_per_w = n // NW
    iters = rows_per_w // CHUNK
    u_per_w = rows_per_w // 4
    slab_rows = u_per_w // 16
    mesh = plsc.VectorSubcoreMesh(core_axis_name="c", subcore_axis_name="s")

    @functools.partial(
        pl.kernel,
        mesh=mesh,
        compiler_params=pltpu.CompilerParams(
            use_tc_tiling_on_sc=False, needs_layout_passes=False),
        out_type=jax.ShapeDtypeStruct((n, C), jnp.float32),
        scratch_types=[
            pltpu.VMEM((4 * slab_rows, 16), jnp.int32),
            [[pltpu.VMEM((G,), jnp.int32) for _ in range(CH)]
             for _ in range(NBUF)],
            [pltpu.VMEM((CHUNK, C), jnp.float32) for _ in range(NBUF)],
            pltpu.SemaphoreType.DMA,
            [pltpu.SemaphoreType.DMA for _ in range(NBUF)],
            [pltpu.SemaphoreType.DMA for _ in range(NBUF)],
        ],
    )
    def scatter_kernel(xt_hbm, idx_hbm, out_hbm, idx_v, i_bufs, row_bufs,
                       isem, lsem, ssem):
        wid = lax.axis_index("s") * 2 + lax.axis_index("c")
        base = wid * rows_per_w

        for q in range(4):
            src_row = pl.multiple_of(
                q * (n4 // 16) + wid * slab_rows, slab_rows)
            pltpu.async_copy(
                idx_hbm.at[pl.ds(src_row, slab_rows)],
                idx_v.at[pl.ds(q * slab_rows, slab_rows)],
                isem,
            )
        pltpu.make_async_copy(
            idx_hbm.at[pl.ds(0, 4 * slab_rows)], idx_v, isem
        ).wait()

        lane = lax.iota(jnp.int32, 16)

        def issue_load(it, b):
            rbase = pl.multiple_of(base + it * CHUNK, CHUNK)
            pltpu.async_copy(xt_hbm.at[pl.ds(rbase, CHUNK)], row_bufs[b], lsem[b])

        for b in range(NBUF):
            issue_load(b, b)

        def outer(i2, carry):
            for b in range(NBUF):
                it = i2 * NBUF + b
                for g in range(CH):
                    gi = it * CH + g
                    for q in range(4):
                        for h in range(2):
                            p = idx_v[slab_rows * q + 2 * gi + h, :]
                            lp = P.bit_length() - 1
                            wv = (
                                (p & ~(4 * P - 1))
                                | ((p & (P - 1)) * 4)
                                | (lax.shift_right_logical(p, lp) & 3)
                            )
                            plsc.store_scatter(
                                i_bufs[b][g], [lane * 4 + (64 * h + q)], wv
                            )
                pltpu.make_async_copy(
                    xt_hbm.at[pl.ds(0, CHUNK)], row_bufs[b], lsem[b]
                ).wait()
                descs = [
                    pltpu.async_copy(
                        row_bufs[b].at[pl.ds(g * G, G)],
                        out_hbm.at[i_bufs[b][g]],
                        ssem[b],
                    )
                    for g in range(CH)
                ]
                for d in descs:
                    d.wait()

                @pl.when(it + NBUF < iters)
                def _():
                    issue_load(it + NBUF, b)

            return carry

        lax.fori_loop(0, iters // NBUF, outer, 0)

    return scatter_kernel(xt, idx16)


def kernel(x, indices):
    b, c, n = x.shape
    x2d = x.reshape(c, n)
    idx16 = indices.reshape(n // 16, 16)
    xt4 = _transpose_to_rows(x2d, n)
    out_t = _sc_scatter_rows(xt4.reshape(n, C), idx16, n)
    out2d = _transpose_to_cols(out_t.reshape(n // 4, 4 * C), n)
    return out2d.reshape(b, c, n)


# speedup vs baseline: 16.4942x; 1.1835x over previous
import functools

import jax
import jax.numpy as jnp
from jax import lax
from jax.experimental import pallas as pl
from jax.experimental.pallas import tpu as pltpu
from jax.experimental.pallas import tpu_sc as plsc

C = 32
NW = 32
G = 128
CH = 8
CHUNK = CH * G
NBUF = 2
BKQ = 4096


def _eye128():
    r = lax.broadcasted_iota(jnp.int32, (4 * C, 4 * C), 0)
    c = lax.broadcasted_iota(jnp.int32, (4 * C, 4 * C), 1)
    return (r == c).astype(jnp.float32)


def _transpose_to_rows(x2d, n):
    n4 = n // 4
    nb = n4 // BKQ

    def body(x0, x1, x2, x3, o_ref):
        x = jnp.concatenate([x0[...], x1[...], x2[...], x3[...]], axis=0)
        o_ref[...] = lax.dot_general(
            x, _eye128(), (((0,), (0,)), ((), ())),
            preferred_element_type=jnp.float32)

    return pl.pallas_call(
        body,
        grid=(nb,),
        in_specs=[
            pl.BlockSpec((C, BKQ), lambda i, k=k: (0, k * nb + i))
            for k in range(4)
        ],
        out_specs=pl.BlockSpec((BKQ, 4 * C), lambda i: (i, 0)),
        out_shape=jax.ShapeDtypeStruct((n4, 4 * C), jnp.float32),
    )(x2d, x2d, x2d, x2d)


P = 8192


def _transpose_to_cols(ot4, n):
    n4 = n // 4
    nb = n4 // P

    def body(x_ref, o_ref):
        r = lax.broadcasted_iota(jnp.int32, (C, 4 * C), 0)
        l = lax.broadcasted_iota(jnp.int32, (C, 4 * C), 1)
        for k in range(4):
            sel = (l == C * k + r).astype(jnp.float32)
            o_ref[:, P * k:P * (k + 1)] = lax.dot_general(
                sel, x_ref[...], (((1,), (1,)), ((), ())),
                preferred_element_type=jnp.float32)

    return pl.pallas_call(
        body,
        grid=(nb,),
        in_specs=[pl.BlockSpec((P, 4 * C), lambda i: (i, 0))],
        out_specs=pl.BlockSpec((C, 4 * P), lambda i: (0, i)),
        out_shape=jax.ShapeDtypeStruct((C, n), jnp.float32),
    )(ot4)


def _sc_scatter_rows(xt, idx16, n):
    n4 = n // 4
    rows_per_w = n // NW
    iters = rows_per_w // CHUNK
    u_per_w = rows_per_w // 4
    slab_rows = u_per_w // 16
    mesh = plsc.VectorSubcoreMesh(core_axis_name="c", subcore_axis_name="s")

    @functools.partial(
        pl.kernel,
        mesh=mesh,
        compiler_params=pltpu.CompilerParams(
            use_tc_tiling_on_sc=False, needs_layout_passes=False),
        out_type=jax.ShapeDtypeStruct((n, C), jnp.float32),
        scratch_types=[
            pltpu.VMEM((4 * slab_rows, 16), jnp.int32),
            [[pltpu.VMEM((G,), jnp.int32) for _ in range(CH)]
             for _ in range(NBUF)],
            [pltpu.VMEM((CHUNK, C), jnp.float32) for _ in range(NBUF)],
            pltpu.SemaphoreType.DMA,
            [pltpu.SemaphoreType.DMA for _ in range(NBUF)],
            [pltpu.SemaphoreType.DMA for _ in range(NBUF)],
        ],
    )
    def scatter_kernel(xt_hbm, idx_hbm, out_hbm, idx_v, i_bufs, row_bufs,
                       isem, lsem, ssem):
        wid = lax.axis_index("s") * 2 + lax.axis_index("c")
        base = wid * rows_per_w

        for q in range(4):
            src_row = pl.multiple_of(
                q * (n4 // 16) + wid * slab_rows, slab_rows)
            pltpu.async_copy(
                idx_hbm.at[pl.ds(src_row, slab_rows)],
                idx_v.at[pl.ds(q * slab_rows, slab_rows)],
                isem,
            )
        pltpu.make_async_copy(
            idx_hbm.at[pl.ds(0, 4 * slab_rows)], idx_v, isem
        ).wait()

        lane = lax.iota(jnp.int32, 16)

        def issue_load(it, b):
            rbase = pl.multiple_of(base + it * CHUNK, CHUNK)
            pltpu.async_copy(xt_hbm.at[pl.ds(rbase, CHUNK)], row_bufs[b], lsem[b])

        for b in range(NBUF):
            issue_load(b, b)

        def outer(i2, carry):
            for b in range(NBUF):
                it = i2 * NBUF + b
                for g in range(CH):
                    gi = it * CH + g
                    for q in range(4):
                        for h in range(2):
                            p = idx_v[slab_rows * q + 2 * gi + h, :]
                            lp = P.bit_length() - 1
                            wv = (
                                (p & ~(4 * P - 1))
                                | ((p & (P - 1)) * 4)
                                | (lax.shift_right_logical(p, lp) & 3)
                            )
                            plsc.store_scatter(
                                i_bufs[b][g], [lane * 4 + (64 * h + q)], wv
                            )
                pltpu.make_async_copy(
                    xt_hbm.at[pl.ds(0, CHUNK)], row_bufs[b], lsem[b]
                ).wait()
                descs = [
                    pltpu.async_copy(
                        row_bufs[b].at[pl.ds(g * G, G)],
                        out_hbm.at[i_bufs[b][g]],
                        ssem[b],
                    )
                    for g in range(CH)
                ]
                for d in descs:
                    d.wait()

                @pl.when(it + NBUF < iters)
                def _():
                    issue_load(it + NBUF, b)

            return carry

        lax.fori_loop(0, iters // NBUF, outer, 0)

    return scatter_kernel(xt, idx16)


def kernel(x, indices):
    b, c, n = x.shape
    x2d = x.reshape(c, n)
    idx16 = indices.reshape(n // 16, 16)
    xt4 = _transpose_to_rows(x2d, n)
    out_t = _sc_scatter_rows(xt4.reshape(n, C), idx16, n)
    out2d = _transpose_to_cols(out_t.reshape(n // 4, 4 * C), n)
    return out2d.reshape(b, c, n)


# speedup vs baseline: 17.6861x; 1.0723x over previous
import functools

import jax
import jax.numpy as jnp
from jax import lax
from jax.experimental import pallas as pl
from jax.experimental.pallas import tpu as pltpu
from jax.experimental.pallas import tpu_sc as plsc

C = 32
NW = 32
G = 128
CH = 8
CHUNK = CH * G
NBUF = 2
BKQ = 8192


def _eye128():
    r = lax.broadcasted_iota(jnp.int32, (4 * C, 4 * C), 0)
    c = lax.broadcasted_iota(jnp.int32, (4 * C, 4 * C), 1)
    return (r == c).astype(jnp.float32)


def _transpose_to_rows(x2d, n):
    n4 = n // 4
    nb = n4 // BKQ

    def body(x0, x1, x2, x3, o_ref):
        x = jnp.concatenate([x0[...], x1[...], x2[...], x3[...]], axis=0)
        o_ref[...] = lax.dot_general(
            x, _eye128(), (((0,), (0,)), ((), ())),
            preferred_element_type=jnp.float32)

    return pl.pallas_call(
        body,
        grid=(nb,),
        in_specs=[
            pl.BlockSpec((C, BKQ), lambda i, k=k: (0, k * nb + i))
            for k in range(4)
        ],
        out_specs=pl.BlockSpec((BKQ, 4 * C), lambda i: (i, 0)),
        out_shape=jax.ShapeDtypeStruct((n4, 4 * C), jnp.float32),
    )(x2d, x2d, x2d, x2d)


P = 16384


def _transpose_to_cols(ot4, n):
    n4 = n // 4
    nb = n4 // P

    def body(x_ref, o_ref):
        r = lax.broadcasted_iota(jnp.int32, (C, 4 * C), 0)
        l = lax.broadcasted_iota(jnp.int32, (C, 4 * C), 1)
        for k in range(4):
            sel = (l == C * k + r).astype(jnp.float32)
            o_ref[:, P * k:P * (k + 1)] = lax.dot_general(
                sel, x_ref[...], (((1,), (1,)), ((), ())),
                preferred_element_type=jnp.float32)

    return pl.pallas_call(
        body,
        grid=(nb,),
        in_specs=[pl.BlockSpec((P, 4 * C), lambda i: (i, 0))],
        out_specs=pl.BlockSpec((C, 4 * P), lambda i: (0, i)),
        out_shape=jax.ShapeDtypeStruct((C, n), jnp.float32),
    )(ot4)


def _sc_scatter_rows(xt, idx16, n):
    n4 = n // 4
    rows_per_w = n // NW
    iters = rows_per_w // CHUNK
    u_per_w = rows_per_w // 4
    slab_rows = u_per_w // 16
    mesh = plsc.VectorSubcoreMesh(core_axis_name="c", subcore_axis_name="s")

    @functools.partial(
        pl.kernel,
        mesh=mesh,
        compiler_params=pltpu.CompilerParams(
            use_tc_tiling_on_sc=False, needs_layout_passes=False),
        out_type=jax.ShapeDtypeStruct((n, C), jnp.float32),
        scratch_types=[
            pltpu.VMEM((4 * slab_rows, 16), jnp.int32),
            [[pltpu.VMEM((G,), jnp.int32) for _ in range(CH)]
             for _ in range(NBUF)],
            [pltpu.VMEM((CHUNK, C), jnp.float32) for _ in range(NBUF)],
            pltpu.SemaphoreType.DMA,
            [pltpu.SemaphoreType.DMA for _ in range(NBUF)],
            [pltpu.SemaphoreType.DMA for _ in range(NBUF)],
        ],
    )
    def scatter_kernel(xt_hbm, idx_hbm, out_hbm, idx_v, i_bufs, row_bufs,
                       isem, lsem, ssem):
        wid = lax.axis_index("s") * 2 + lax.axis_index("c")
        base = wid * rows_per_w

        for q in range(4):
            src_row = pl.multiple_of(
                q * (n4 // 16) + wid * slab_rows, slab_rows)
            pltpu.async_copy(
                idx_hbm.at[pl.ds(src_row, slab_rows)],
                idx_v.at[pl.ds(q * slab_rows, slab_rows)],
                isem,
            )
        pltpu.make_async_copy(
            idx_hbm.at[pl.ds(0, 4 * slab_rows)], idx_v, isem
        ).wait()

        lane = lax.iota(jnp.int32, 16)

        def issue_load(it, b):
            rbase = pl.multiple_of(base + it * CHUNK, CHUNK)
            pltpu.async_copy(xt_hbm.at[pl.ds(rbase, CHUNK)], row_bufs[b], lsem[b])

        for b in range(NBUF):
            issue_load(b, b)

        def outer(i2, carry):
            for b in range(NBUF):
                it = i2 * NBUF + b
                for g in range(CH):
                    gi = it * CH + g
                    for q in range(4):
                        for h in range(2):
                            p = idx_v[slab_rows * q + 2 * gi + h, :]
                            lp = P.bit_length() - 1
                            wv = (
                                (p & ~(4 * P - 1))
                                | ((p & (P - 1)) * 4)
                                | (lax.shift_right_logical(p, lp) & 3)
                            )
                            plsc.store_scatter(
                                i_bufs[b][g], [lane * 4 + (64 * h + q)], wv
                            )
                pltpu.make_async_copy(
                    xt_hbm.at[pl.ds(0, CHUNK)], row_bufs[b], lsem[b]
                ).wait()
                descs = [
                    pltpu.async_copy(
                        row_bufs[b].at[pl.ds(g * G, G)],
                        out_hbm.at[i_bufs[b][g]],
                        ssem[b],
                    )
                    for g in range(CH)
                ]
                for d in descs:
                    d.wait()

                @pl.when(it + NBUF < iters)
                def _():
                    issue_load(it + NBUF, b)

            return carry

        lax.fori_loop(0, iters // NBUF, outer, 0)

    return scatter_kernel(xt, idx16)


def kernel(x, indices):
    b, c, n = x.shape
    x2d = x.reshape(c, n)
    idx16 = indices.reshape(n // 16, 16)
    xt4 = _transpose_to_rows(x2d, n)
    out_t = _sc_scatter_rows(xt4.reshape(n, C), idx16, n)
    out2d = _transpose_to_cols(out_t.reshape(n // 4, 4 * C), n)
    return out2d.reshape(b, c, n)
